# Initial kernel scaffold; baseline (speedup 1.0000x reference)
import jax
import jax.numpy as jnp
from jax.experimental import pallas as pl


def kernel(feats, edge_index, temp):
    raise NotImplementedError("write your pallas kernel here")


# speedup vs baseline: 3.9477x; 3.9477x over previous
import jax
import jax.numpy as jnp
from jax import lax
from jax.experimental import pallas as pl
from jax.experimental.pallas import tpu as pltpu
from jax.experimental.pallas import tpu_sc as plsc

N = 10000
NP_ = 10240
E = 160000
D = 256
K = 10
COLS = 64
NSUB = 16
EPT = E // NSUB
BB = 125
NB = EPT // BB
NPT = NP_ // NSUB
RC = 128
NRC = NPT // RC
ZR = 64
RV = COLS // 16


def _rsqrt16(d):
    y = jnp.full((16,), 64.0, jnp.float32)
    for _ in range(12):
        y = jnp.float32(0.5) * (y + d / y)
    v = jnp.float32(1.0) / y
    return jnp.where(d == jnp.float32(0.0), jnp.float32(jnp.inf), v)


def _gpr_body(feats, src4, dst_rs, temp32, out,
              s_tab, y_acc, deg16,
              src_v, dst_v, rows, fin, zbuf, ones_v, dchunk, temp_v):
    c = lax.axis_index("c")
    sid = lax.axis_index("s")
    nbase = sid * NPT

    zero = jnp.zeros((16,), jnp.float32)
    one = jnp.ones((16,), jnp.float32)

    def _zrow(i, carry):
        for u in range(RV):
            zbuf[i, pl.ds(16 * u, 16)] = zero
        return carry
    lax.fori_loop(0, ZR, _zrow, 0)

    def _orow(i, carry):
        ones_v[i] = one
        return carry
    lax.fori_loop(0, BB, _orow, 0)

    def _drow(i, carry):
        dchunk[i] = zero
        return carry
    lax.fori_loop(0, NPT, _drow, 0)

    pltpu.sync_copy(temp32, temp_v)
    pltpu.sync_copy(src4.at[c, sid], src_v)
    pltpu.sync_copy(dst_rs.at[sid], dst_v)

    pltpu.sync_copy(dchunk, deg16.at[pl.ds(nbase, NPT)])
    plsc.subcore_barrier()
    def _degj(j, carry):
        pltpu.sync_copy(ones_v, deg16.at[dst_v.at[j]], add=True)
        return carry
    lax.fori_loop(0, NB, _degj, 0)
    plsc.subcore_barrier()
    pltpu.sync_copy(deg16.at[pl.ds(nbase, NPT)], dchunk)

    def _nrm(i, carry):
        dchunk[i] = _rsqrt16(dchunk[i])
        return carry
    lax.fori_loop(0, NPT, _nrm, 0)

    t0 = temp_v[pl.ds(0, 16)][0]
    for h in range(2):
        boff = (2 * c + h) * NP_
        for p in range(NRC):
            rbase = nbase + RC * p
            pltpu.sync_copy(
                feats.at[pl.ds(rbase, RC), pl.ds(c * 128 + h * COLS, COLS)],
                fin.at[pl.ds(0, RC)])
            def _irow(i, carry):
                nrm = dchunk[RC * p + i]
                for u in range(RV):
                    x = fin[i, pl.ds(16 * u, 16)]
                    fin[RC + i, pl.ds(16 * u, 16)] = x * nrm
                    fin[i, pl.ds(16 * u, 16)] = x * t0
                return carry
            lax.fori_loop(0, RC, _irow, 0)
            pltpu.sync_copy(fin.at[pl.ds(RC, RC)],
                            s_tab.at[pl.ds(boff + rbase, RC)])
            pltpu.sync_copy(
                fin.at[pl.ds(0, RC)],
                out.at[pl.ds(rbase, RC), pl.ds(c * 128 + h * COLS, COLS)])
    for z in range(NPT // ZR):
        pltpu.sync_copy(zbuf, y_acc.at[pl.ds(nbase + ZR * z, ZR)])

    def _hop(k, carry):
        gamma = temp_v[pl.ds(k + 1, 16)][0]
        for h in range(2):
            boff = (2 * c + h) * NP_
            plsc.subcore_barrier()
            def _edge(j, ecarry):
                pltpu.sync_copy(s_tab.at[src_v.at[h, j]], rows)
                pltpu.sync_copy(rows, y_acc.at[dst_v.at[j]], add=True)
                return ecarry
            lax.fori_loop(0, NB, _edge, 0)
            plsc.subcore_barrier()
            for p in range(NRC):
                rbase = nbase + RC * p
                pltpu.sync_copy(y_acc.at[pl.ds(rbase, RC)],
                                fin.at[pl.ds(0, RC)])
                for z in range(RC // ZR):
                    pltpu.sync_copy(
                        zbuf, y_acc.at[pl.ds(rbase + ZR * z, ZR)])
                pltpu.sync_copy(
                    out.at[pl.ds(rbase, RC),
                           pl.ds(c * 128 + h * COLS, COLS)],
                    fin.at[pl.ds(RC, RC)])
                def _frow(i, fcarry):
                    nrm = dchunk[RC * p + i]
                    for u in range(RV):
                        y = fin[i, pl.ds(16 * u, 16)]
                        xp = y * nrm
                        fin[RC + i, pl.ds(16 * u, 16)] = (
                            fin[RC + i, pl.ds(16 * u, 16)] + gamma * xp)
                        fin[i, pl.ds(16 * u, 16)] = xp * nrm
                    return fcarry
                lax.fori_loop(0, RC, _frow, 0)
                pltpu.sync_copy(
                    fin.at[pl.ds(RC, RC)],
                    out.at[pl.ds(rbase, RC),
                           pl.ds(c * 128 + h * COLS, COLS)])
                pltpu.sync_copy(fin.at[pl.ds(0, RC)],
                                s_tab.at[pl.ds(boff + rbase, RC)])
        return carry
    lax.fori_loop(0, K, _hop, 0)


_gpr = pl.kernel(
    _gpr_body,
    out_type=jax.ShapeDtypeStruct((NP_, D), jnp.float32),
    mesh=plsc.VectorSubcoreMesh(core_axis_name="c", subcore_axis_name="s"),
    compiler_params=pltpu.CompilerParams(use_tc_tiling_on_sc=False),
    scratch_types=[
        pltpu.HBM((4 * NP_, COLS), jnp.float32),
        pltpu.VMEM_SHARED((NP_, COLS), jnp.float32),
        pltpu.VMEM_SHARED((NP_, 16), jnp.float32),
        pltpu.VMEM((2, NB, BB), jnp.int32),
        pltpu.VMEM((NB, BB), jnp.int32),
        pltpu.VMEM((BB, COLS), jnp.float32),
        pltpu.VMEM((2 * RC, COLS), jnp.float32),
        pltpu.VMEM((ZR, COLS), jnp.float32),
        pltpu.VMEM((BB, 16), jnp.float32),
        pltpu.VMEM((NPT, 16), jnp.float32),
        pltpu.VMEM((32,), jnp.float32),
    ],
)


def kernel(feats, edge_index, temp):
    src = edge_index[0].reshape(NSUB, NB, BB)
    dst = edge_index[1].reshape(NSUB, NB, BB)
    src4 = jnp.stack([jnp.stack([src, src + NP_]),
                      jnp.stack([src + 2 * NP_, src + 3 * NP_])])
    src4 = src4.transpose(0, 2, 1, 3, 4)
    temp32 = jnp.zeros((32,), jnp.float32).at[: K + 1].set(temp)
    feats_p = jnp.zeros((NP_, D), jnp.float32).at[:N].set(feats)
    return _gpr(feats_p, src4, dst, temp32)[:N]


# speedup vs baseline: 4.9419x; 1.2518x over previous
import jax
import jax.numpy as jnp
from jax import lax
from jax.experimental import pallas as pl
from jax.experimental.pallas import tpu as pltpu
from jax.experimental.pallas import tpu_sc as plsc

N = 10000
NP_ = 10240
E = 160000
D = 256
K = 10
COLS = 64
NSUB = 16
EPT = E // NSUB
BB = 125
NB = EPT // BB
NPT = NP_ // NSUB
RC = 128
NRC = NPT // RC
ZR = 64
RV = COLS // 16


def _rsqrt16(d):
    y = jnp.full((16,), 64.0, jnp.float32)
    for _ in range(12):
        y = jnp.float32(0.5) * (y + d / y)
    v = jnp.float32(1.0) / y
    return jnp.where(d == jnp.float32(0.0), jnp.float32(jnp.inf), v)


def _gpr_body(feats, src4, dst_rs, temp32, out,
              s_tab, y_acc, deg16,
              src_v, dst_v, rows, fin, zbuf, ones_v, dchunk, temp_v,
              gsem, ssem):
    c = lax.axis_index("c")
    sid = lax.axis_index("s")
    nbase = sid * NPT

    zero = jnp.zeros((16,), jnp.float32)
    one = jnp.ones((16,), jnp.float32)

    def _zrow(i, carry):
        for u in range(RV):
            zbuf[i, pl.ds(16 * u, 16)] = zero
        return carry
    lax.fori_loop(0, ZR, _zrow, 0)

    def _orow(i, carry):
        ones_v[i] = one
        return carry
    lax.fori_loop(0, BB, _orow, 0)

    def _drow(i, carry):
        dchunk[i] = zero
        return carry
    lax.fori_loop(0, NPT, _drow, 0)

    pltpu.sync_copy(temp32, temp_v)
    pltpu.sync_copy(src4.at[c, sid], src_v)
    pltpu.sync_copy(dst_rs.at[sid], dst_v)

    pltpu.sync_copy(dchunk, deg16.at[pl.ds(nbase, NPT)])
    plsc.subcore_barrier()
    def _degj(j, carry):
        pltpu.sync_copy(ones_v, deg16.at[dst_v.at[j]], add=True)
        return carry
    lax.fori_loop(0, NB, _degj, 0)
    plsc.subcore_barrier()
    pltpu.sync_copy(deg16.at[pl.ds(nbase, NPT)], dchunk)

    def _nrm(i, carry):
        dchunk[i] = _rsqrt16(dchunk[i])
        return carry
    lax.fori_loop(0, NPT, _nrm, 0)

    t0 = temp_v[pl.ds(0, 16)][0]
    for h in range(2):
        boff = (2 * c + h) * NP_
        for p in range(NRC):
            rbase = nbase + RC * p
            pltpu.sync_copy(
                feats.at[pl.ds(rbase, RC), pl.ds(c * 128 + h * COLS, COLS)],
                fin.at[pl.ds(0, RC)])
            def _irow(i, carry):
                nrm = dchunk[RC * p + i]
                for u in range(RV):
                    x = fin[i, pl.ds(16 * u, 16)]
                    fin[RC + i, pl.ds(16 * u, 16)] = x * nrm
                    fin[i, pl.ds(16 * u, 16)] = x * t0
                return carry
            lax.fori_loop(0, RC, _irow, 0)
            pltpu.sync_copy(fin.at[pl.ds(RC, RC)],
                            s_tab.at[pl.ds(boff + rbase, RC)])
            pltpu.sync_copy(
                fin.at[pl.ds(0, RC)],
                out.at[pl.ds(rbase, RC), pl.ds(c * 128 + h * COLS, COLS)])
    for z in range(NPT // ZR):
        pltpu.sync_copy(zbuf, y_acc.at[pl.ds(nbase + ZR * z, ZR)])

    def _hop(k, carry):
        gamma = temp_v[pl.ds(k + 1, 16)][0]
        for h in range(2):
            boff = (2 * c + h) * NP_
            plsc.subcore_barrier()
            pltpu.async_copy(s_tab.at[src_v.at[h, 0]], rows.at[0], gsem)

            @pl.loop(0, NB, step=2)
            def _edge(j):
                for b in range(2):
                    jj = j + b
                    pltpu.make_async_copy(
                        s_tab.at[src_v.at[h, jj]], rows.at[b], gsem).wait()

                    @pl.when(jj >= 1)
                    def _():
                        pltpu.make_async_copy(
                            rows.at[1 - b], y_acc.at[dst_v.at[jj]],
                            ssem).wait()

                    @pl.when(jj + 1 < NB)
                    def _():
                        pltpu.async_copy(
                            s_tab.at[src_v.at[h, jj + 1]], rows.at[1 - b],
                            gsem)

                    pltpu.async_copy(
                        rows.at[b], y_acc.at[dst_v.at[jj]], ssem, add=True)

            pltpu.make_async_copy(
                rows.at[1], y_acc.at[dst_v.at[0]], ssem).wait()
            plsc.subcore_barrier()
            for p in range(NRC):
                rbase = nbase + RC * p
                pltpu.sync_copy(y_acc.at[pl.ds(rbase, RC)],
                                fin.at[pl.ds(0, RC)])
                for z in range(RC // ZR):
                    pltpu.sync_copy(
                        zbuf, y_acc.at[pl.ds(rbase + ZR * z, ZR)])
                pltpu.sync_copy(
                    out.at[pl.ds(rbase, RC),
                           pl.ds(c * 128 + h * COLS, COLS)],
                    fin.at[pl.ds(RC, RC)])
                def _frow(i, fcarry):
                    nrm = dchunk[RC * p + i]
                    for u in range(RV):
                        y = fin[i, pl.ds(16 * u, 16)]
                        xp = y * nrm
                        fin[RC + i, pl.ds(16 * u, 16)] = (
                            fin[RC + i, pl.ds(16 * u, 16)] + gamma * xp)
                        fin[i, pl.ds(16 * u, 16)] = xp * nrm
                    return fcarry
                lax.fori_loop(0, RC, _frow, 0)
                pltpu.sync_copy(
                    fin.at[pl.ds(RC, RC)],
                    out.at[pl.ds(rbase, RC),
                           pl.ds(c * 128 + h * COLS, COLS)])
                pltpu.sync_copy(fin.at[pl.ds(0, RC)],
                                s_tab.at[pl.ds(boff + rbase, RC)])
        return carry
    lax.fori_loop(0, K, _hop, 0)


_gpr = pl.kernel(
    _gpr_body,
    out_type=jax.ShapeDtypeStruct((NP_, D), jnp.float32),
    mesh=plsc.VectorSubcoreMesh(core_axis_name="c", subcore_axis_name="s"),
    compiler_params=pltpu.CompilerParams(use_tc_tiling_on_sc=False),
    scratch_types=[
        pltpu.HBM((4 * NP_, COLS), jnp.float32),
        pltpu.VMEM_SHARED((NP_, COLS), jnp.float32),
        pltpu.VMEM_SHARED((NP_, 16), jnp.float32),
        pltpu.VMEM((2, NB, BB), jnp.int32),
        pltpu.VMEM((NB, BB), jnp.int32),
        pltpu.VMEM((2, BB, COLS), jnp.float32),
        pltpu.VMEM((2 * RC, COLS), jnp.float32),
        pltpu.VMEM((ZR, COLS), jnp.float32),
        pltpu.VMEM((BB, 16), jnp.float32),
        pltpu.VMEM((NPT, 16), jnp.float32),
        pltpu.VMEM((32,), jnp.float32),
        pltpu.SemaphoreType.DMA,
        pltpu.SemaphoreType.DMA,
    ],
)


def kernel(feats, edge_index, temp):
    src = edge_index[0].reshape(NSUB, NB, BB)
    dst = edge_index[1].reshape(NSUB, NB, BB)
    src4 = jnp.stack([jnp.stack([src, src + NP_]),
                      jnp.stack([src + 2 * NP_, src + 3 * NP_])])
    src4 = src4.transpose(0, 2, 1, 3, 4)
    temp32 = jnp.zeros((32,), jnp.float32).at[: K + 1].set(temp)
    feats_p = jnp.zeros((NP_, D), jnp.float32).at[:N].set(feats)
    return _gpr(feats_p, src4, dst, temp32)[:N]
